# baseline (speedup 1.0000x reference)
import jax
import jax.numpy as jnp
from jax import lax
from jax.experimental import pallas as pl
from jax.experimental.pallas import tpu as pltpu
from jax.experimental.pallas import tpu_sc as plsc

_B = 16384
_F = 26
_V = 100000
_VP = 102400
_D = 50
_C = 13
_FP = 32

_NC = 2
_NS = 16
_L = 16
_NW = _NC * _NS
_BPW = _B // _NW
_G = _BPW // _L

_VB = 51200
_NVB = _VP // _VB


def _tc_body(tab_ref, wf_ref, out_ref):
    f = pl.program_id(0)
    out_ref[...] = jnp.dot(wf_ref[pl.ds(f, 1), :], tab_ref[0],
                           preferred_element_type=jnp.float32)[0]


def _sc_body(s_hbm, xcat_hbm, xc_hbm, wc_hbm, out_hbm,
             xcat_v, idx_v, val_v, acc_v, cont_v, wc_v, sem):
    wid = lax.axis_index("s") * _NC + lax.axis_index("c")
    base = pl.multiple_of(wid * _BPW, _BPW)

    lane = lax.iota(jnp.int32, _L)

    pltpu.sync_copy(wc_hbm, wc_v)
    pltpu.sync_copy(xc_hbm.at[pl.ds(pl.multiple_of(wid * (_BPW * _C), 8),
                                    _BPW * _C)], cont_v)

    pltpu.sync_copy(xcat_hbm.at[wid], xcat_v)
    for f in range(_F):
        fcol = jnp.full((_L,), f, jnp.int32)
        for g in range(_G):
            idx_v[f, pl.ds(g * _L, _L)] = plsc.load_gather(
                xcat_v, [lane + g * _L, fcol])
    copies = [pltpu.async_copy(s_hbm.at[idx_v.at[f]], val_v.at[f], sem)
              for f in range(_F)]

    bias = plsc.load_gather(wc_v, [jnp.full((_L,), _C, jnp.int32)])
    for g in range(_G):
        acc_v[pl.ds(g * _L, _L)] = bias

    lane_c = lane * _C

    def cont_d(d, carry):
        ws = plsc.load_gather(wc_v, [jnp.full((_L,), d, jnp.int32)])
        for g in range(_G):
            v = plsc.load_gather(cont_v, [lane_c + (g * (_L * _C) + d)])
            plsc.addupdate(acc_v.at[pl.ds(g * _L, _L)], v * ws)
        return carry

    lax.fori_loop(0, _C, cont_d, 0)

    for cp in copies:
        cp.wait()

    def field(f, carry):
        for g in range(_G):
            plsc.addupdate(acc_v.at[pl.ds(g * _L, _L)],
                           val_v[f, pl.ds(g * _L, _L)])
        return carry

    lax.fori_loop(0, _F, field, 0)

    pltpu.sync_copy(acc_v, out_hbm.at[pl.ds(base, _BPW)])


@jax.jit
def _run(tabT, wf, xcat3, xc_flat, wc):
    s = pl.pallas_call(
        _tc_body,
        grid=(_F, _NVB),
        in_specs=[
            pl.BlockSpec((1, _D, _VB), lambda f, j: (f, 0, j)),
            pl.BlockSpec((_F, _D), lambda f, j: (0, 0)),
        ],
        out_specs=pl.BlockSpec((_VB,), lambda f, j: (f * _NVB + j,)),
        out_shape=jax.ShapeDtypeStruct((_F * _VP,), jnp.float32),
    )(tabT, wf)

    k = pl.kernel(
        _sc_body,
        out_type=jax.ShapeDtypeStruct((_B,), jnp.float32),
        mesh=plsc.VectorSubcoreMesh(core_axis_name="c", subcore_axis_name="s",
                                    num_cores=_NC, num_subcores=_NS),
        scratch_types=[
            pltpu.VMEM((_BPW, _FP), jnp.int32),
            pltpu.VMEM((_F, _BPW), jnp.int32),
            pltpu.VMEM((_F, _BPW), jnp.float32),
            pltpu.VMEM((_BPW,), jnp.float32),
            pltpu.VMEM((_BPW * _C,), jnp.float32),
            pltpu.VMEM((_L,), jnp.float32),
            pltpu.SemaphoreType.DMA,
        ],
        compiler_params=pltpu.CompilerParams(
            needs_layout_passes=False, use_tc_tiling_on_sc=False),
    )
    return k(s, xcat3, xc_flat, wc)


def kernel(x_cont, x_cat, tables, W, b):
    tabT = tables.transpose(0, 2, 1)
    wf = W[:_F * _D, 0].reshape(_F, _D)
    wc = jnp.concatenate([W[_F * _D:, 0], b,
                          jnp.zeros((_L - _C - 1,), jnp.float32)])
    xcat3 = jnp.pad(x_cat + jnp.arange(_F, dtype=jnp.int32)[None, :] * _VP,
                    ((0, 0), (0, _FP - _F)))
    xcat3 = xcat3.reshape(_NW, _BPW, _FP)
    xc_flat = x_cont.reshape(-1)
    out = _run(tabT, wf, xcat3, xc_flat, wc)
    return out.reshape(_B, 1)


# speedup vs baseline: 1.0888x; 1.0888x over previous
import jax
import jax.numpy as jnp
from jax import lax
from jax.experimental import pallas as pl
from jax.experimental.pallas import tpu as pltpu
from jax.experimental.pallas import tpu_sc as plsc

_B = 16384
_F = 26
_V = 100000
_VP = 102400
_D = 50
_C = 13

_NC = 2
_NS = 16
_L = 16
_NW = _NC * _NS
_BPW = _B // _NW
_G = _BPW // _L

_VB = 51200
_NVB = _VP // _VB


def _tc_body(tab_ref, wf_ref, out_ref):
    f = pl.program_id(0)
    out_ref[...] = jnp.dot(wf_ref[pl.ds(f, 1), :], tab_ref[0],
                           preferred_element_type=jnp.float32)[0]


def _cont_body(xc_ref, w_ref, b_ref, out_ref):
    out_ref[...] = jnp.dot(xc_ref[...], w_ref[...],
                           preferred_element_type=jnp.float32)[:, 0] + b_ref[0]


def _sc_body(s_hbm, gidx_hbm, cont_hbm, out_hbm, idx_v, val_v, acc_v, sem):
    wid = lax.axis_index("s") * _NC + lax.axis_index("c")
    base = pl.multiple_of(wid * _BPW, _BPW)

    pltpu.sync_copy(gidx_hbm.at[wid], idx_v)
    copies = [pltpu.async_copy(s_hbm.at[idx_v.at[f]], val_v.at[f], sem)
              for f in range(_F)]

    pltpu.sync_copy(cont_hbm.at[pl.ds(base, _BPW)], acc_v)

    for cp in copies:
        cp.wait()

    def field(f, carry):
        for g in range(_G):
            plsc.addupdate(acc_v.at[pl.ds(g * _L, _L)],
                           val_v[f, pl.ds(g * _L, _L)])
        return carry

    lax.fori_loop(0, _F, field, 0)

    pltpu.sync_copy(acc_v, out_hbm.at[pl.ds(base, _BPW)])


@jax.jit
def _run(tabT, wf, gidx, x_cont, wcv, b):
    s = pl.pallas_call(
        _tc_body,
        grid=(_F, _NVB),
        in_specs=[
            pl.BlockSpec((1, _D, _VB), lambda f, j: (f, 0, j)),
            pl.BlockSpec((_F, _D), lambda f, j: (0, 0)),
        ],
        out_specs=pl.BlockSpec((_VB,), lambda f, j: (f * _NVB + j,)),
        out_shape=jax.ShapeDtypeStruct((_F * _VP,), jnp.float32),
    )(tabT, wf)

    cont = pl.pallas_call(
        _cont_body,
        in_specs=[
            pl.BlockSpec((_B, _C), lambda: (0, 0)),
            pl.BlockSpec((_C, 1), lambda: (0, 0)),
            pl.BlockSpec(memory_space=pltpu.SMEM),
        ],
        out_specs=pl.BlockSpec((_B,), lambda: (0,)),
        out_shape=jax.ShapeDtypeStruct((_B,), jnp.float32),
    )(x_cont, wcv, b)

    k = pl.kernel(
        _sc_body,
        out_type=jax.ShapeDtypeStruct((_B,), jnp.float32),
        mesh=plsc.VectorSubcoreMesh(core_axis_name="c", subcore_axis_name="s",
                                    num_cores=_NC, num_subcores=_NS),
        scratch_types=[
            pltpu.VMEM((_F, _BPW), jnp.int32),
            pltpu.VMEM((_F, _BPW), jnp.float32),
            pltpu.VMEM((_BPW,), jnp.float32),
            pltpu.SemaphoreType.DMA,
        ],
        compiler_params=pltpu.CompilerParams(
            needs_layout_passes=False, use_tc_tiling_on_sc=False),
    )
    return k(s, gidx, cont)


def kernel(x_cont, x_cat, tables, W, b):
    tabT = tables.transpose(0, 2, 1)
    wf = W[:_F * _D, 0].reshape(_F, _D)
    wcv = W[_F * _D:, :]
    gidx = (x_cat + jnp.arange(_F, dtype=jnp.int32)[None, :] * _VP)
    gidx = gidx.T.reshape(_F, _NW, _BPW).transpose(1, 0, 2)
    out = _run(tabT, wf, gidx, x_cont, wcv, b)
    return out.reshape(_B, 1)


# speedup vs baseline: 1.1235x; 1.0319x over previous
import jax
import jax.numpy as jnp
from jax import lax
from jax.experimental import pallas as pl
from jax.experimental.pallas import tpu as pltpu
from jax.experimental.pallas import tpu_sc as plsc

_B = 16384
_F = 26
_V = 100000
_VP = 102400
_D = 50
_C = 13

_NC = 2
_NS = 16
_L = 16
_NW = _NC * _NS
_BPW = _B // _NW
_G = _BPW // _L

_VB = 51200
_NVB = _VP // _VB


def _tc_body(tab_ref, wf_ref, xc_ref, wc_ref, b_ref, s_ref, cont_ref):
    f = pl.program_id(0)
    s_ref[...] = jnp.dot(wf_ref[pl.ds(f, 1), :], tab_ref[0],
                         preferred_element_type=jnp.float32)[0]

    @pl.when(jnp.logical_and(f == 0, pl.program_id(1) == 0))
    def _():
        cont_ref[...] = jnp.dot(
            xc_ref[...], wc_ref[...],
            preferred_element_type=jnp.float32)[:, 0] + b_ref[0]


def _sc_body(s_hbm, gidx_hbm, cont_hbm, out_hbm, idx_v, val_v, acc_v, sem):
    wid = lax.axis_index("s") * _NC + lax.axis_index("c")
    base = pl.multiple_of(wid * _BPW, _BPW)

    pltpu.sync_copy(gidx_hbm.at[wid], idx_v)
    copies = [pltpu.async_copy(s_hbm.at[idx_v.at[f]], val_v.at[f], sem)
              for f in range(_F)]

    pltpu.sync_copy(cont_hbm.at[pl.ds(base, _BPW)], acc_v)

    for cp in copies:
        cp.wait()

    def field(f, carry):
        for g in range(_G):
            plsc.addupdate(acc_v.at[pl.ds(g * _L, _L)],
                           val_v[f, pl.ds(g * _L, _L)])
        return carry

    lax.fori_loop(0, _F, field, 0)

    pltpu.sync_copy(acc_v, out_hbm.at[pl.ds(base, _BPW)])


@jax.jit
def _run(tabT, wf, gidx, x_cont, wcv, b):
    s, cont = pl.pallas_call(
        _tc_body,
        grid=(_F, _NVB),
        in_specs=[
            pl.BlockSpec((1, _D, _VB), lambda f, j: (f, 0, j)),
            pl.BlockSpec((_F, _D), lambda f, j: (0, 0)),
            pl.BlockSpec((_B, _C), lambda f, j: (0, 0)),
            pl.BlockSpec((_C, 1), lambda f, j: (0, 0)),
            pl.BlockSpec(memory_space=pltpu.SMEM),
        ],
        out_specs=[
            pl.BlockSpec((_VB,), lambda f, j: (f * _NVB + j,)),
            pl.BlockSpec((_B,), lambda f, j: (0,)),
        ],
        out_shape=[
            jax.ShapeDtypeStruct((_F * _VP,), jnp.float32),
            jax.ShapeDtypeStruct((_B,), jnp.float32),
        ],
    )(tabT, wf, x_cont, wcv, b)

    k = pl.kernel(
        _sc_body,
        out_type=jax.ShapeDtypeStruct((_B,), jnp.float32),
        mesh=plsc.VectorSubcoreMesh(core_axis_name="c", subcore_axis_name="s",
                                    num_cores=_NC, num_subcores=_NS),
        scratch_types=[
            pltpu.VMEM((_F, _BPW), jnp.int32),
            pltpu.VMEM((_F, _BPW), jnp.float32),
            pltpu.VMEM((_BPW,), jnp.float32),
            pltpu.SemaphoreType.DMA,
        ],
        compiler_params=pltpu.CompilerParams(
            needs_layout_passes=False, use_tc_tiling_on_sc=False),
    )
    return k(s, gidx, cont)


def kernel(x_cont, x_cat, tables, W, b):
    tabT = tables.transpose(0, 2, 1)
    wf = W[:_F * _D, 0].reshape(_F, _D)
    wcv = W[_F * _D:, :]
    gidx = (x_cat + jnp.arange(_F, dtype=jnp.int32)[None, :] * _VP)
    gidx = gidx.T.reshape(_F, _NW, _BPW).transpose(1, 0, 2)
    out = _run(tabT, wf, gidx, x_cont, wcv, b)
    return out.reshape(_B, 1)
